# Initial kernel scaffold; baseline (speedup 1.0000x reference)
import jax
import jax.numpy as jnp
from jax.experimental import pallas as pl


def kernel(x, emb_table, pos_table, sos_token):
    raise NotImplementedError("write your pallas kernel here")


# speedup vs baseline: 1.1173x; 1.1173x over previous
import functools

import jax
import jax.numpy as jnp
from jax import lax
from jax.experimental import pallas as pl
from jax.experimental.pallas import tpu as pltpu
from jax.experimental.pallas import tpu_sc as plsc

B = 4
S = 2048
D = 768
L = 16
NC = 2
NS = 16
NW = NC * NS
CHUNK = S // NW


def _emb_body(idx_hbm, emb_hbm, pos_hbm, sos_hbm, out_hbm,
              idx_v, rows_v, pos_v, sos_v, sem):
    wid = lax.axis_index("s") * NC + lax.axis_index("c")
    base = wid * CHUNK

    pltpu.sync_copy(pos_hbm.at[pl.ds(base, CHUNK), :], pos_v)

    @pl.when(wid == 0)
    def _load_sos():
        pltpu.sync_copy(sos_hbm, sos_v)

    def _add_pos_row(r, _):
        for j in range(D // L):
            sl = pl.ds(j * L, L)
            plsc.addupdate(rows_v.at[r, sl], pos_v[r, sl])
        return _

    for b in range(B):
        pltpu.sync_copy(idx_hbm.at[pl.ds(b * S + base, CHUNK)], idx_v)
        pltpu.async_copy(emb_hbm.at[idx_v], rows_v, sem).wait()
        lax.fori_loop(0, CHUNK, _add_pos_row, 0, unroll=False)

        @pl.when(wid == 0)
        def _fix_sos_row():
            for j in range(D // L):
                sl = pl.ds(j * L, L)
                rows_v[0, sl] = sos_v[sl] + pos_v[0, sl]

        pltpu.sync_copy(rows_v, out_hbm.at[b, pl.ds(base, CHUNK), :])


@functools.partial(jax.jit, static_argnames=())
def _run(idx_flat, emb_table, pos_table, sos_token):
    mesh = plsc.VectorSubcoreMesh(core_axis_name="c", subcore_axis_name="s")
    f = pl.kernel(
        _emb_body,
        out_type=jax.ShapeDtypeStruct((B, S, D), jnp.float32),
        mesh=mesh,
        scratch_types=[
            pltpu.VMEM((CHUNK,), jnp.int32),
            pltpu.VMEM((CHUNK, D), jnp.float32),
            pltpu.VMEM((CHUNK, D), jnp.float32),
            pltpu.VMEM((D,), jnp.float32),
            pltpu.SemaphoreType.DMA,
        ],
    )
    return f(idx_flat, emb_table, pos_table, sos_token)


def kernel(x, emb_table, pos_table, sos_token):
    idx = jnp.concatenate(
        [jnp.zeros((B, 1), jnp.int32), x[:, :-1].astype(jnp.int32)], axis=1
    ).reshape(-1)
    return _run(idx, emb_table, pos_table, sos_token)
